# baseline (speedup 1.0000x reference)
import functools

import jax
import jax.numpy as jnp
from jax import lax
from jax.experimental import pallas as pl
from jax.experimental.pallas import tpu as pltpu
from jax.experimental.pallas import tpu_sc as plsc

_B = 2
_D = 1024
_S = 8192
_NWD = 8
_NWS = 4
_DW = _D // _NWD
_SW = _S // _NWS
_SC = 128
_NCHUNK = _SW // _SC
_NG = _SC // 16
_NSLOT = 5


def _sc_body(x_hbm, pe_hbm, o_hbm, pe_v, x_v, sem_pe, sem_x, sem_out):
    wid = lax.axis_index("s") * 2 + lax.axis_index("c")
    d0 = (wid % _NWD) * _DW
    s_base = (wid // _NWD) * _SW
    iota = lax.iota(jnp.int32, 16)

    def fill(c):
        s0 = s_base + c * _SC
        pslot = c % 2
        xs0, xs1 = (2 * c) % _NSLOT, (2 * c + 1) % _NSLOT
        return (
            pltpu.async_copy(pe_hbm.at[pl.ds(s0, _SC), pl.ds(d0, _DW)],
                             pe_v.at[pslot], sem_pe[pslot]),
            pltpu.async_copy(x_hbm.at[0, pl.ds(d0, _DW), pl.ds(s0, _SC)],
                             x_v.at[xs0], sem_x[xs0]),
            pltpu.async_copy(x_hbm.at[1, pl.ds(d0, _DW), pl.ds(s0, _SC)],
                             x_v.at[xs1], sem_x[xs1]),
        )

    def drain(c):
        s0 = s_base + c * _SC
        xs0, xs1 = (2 * c) % _NSLOT, (2 * c + 1) % _NSLOT
        return {
            xs0: pltpu.async_copy(
                x_v.at[xs0], o_hbm.at[0, pl.ds(d0, _DW), pl.ds(s0, _SC)],
                sem_out[xs0]),
            xs1: pltpu.async_copy(
                x_v.at[xs1], o_hbm.at[1, pl.ds(d0, _DW), pl.ds(s0, _SC)],
                sem_out[xs1]),
        }

    pending_fill = {0: fill(0)}
    pending_drain = {}

    for c in range(_NCHUNK):
        if c + 1 < _NCHUNK:
            for slot in ((2 * (c + 1)) % _NSLOT, (2 * (c + 1) + 1) % _NSLOT):
                if slot in pending_drain:
                    pending_drain.pop(slot).wait()
            pending_fill[c + 1] = fill(c + 1)
        for cp in pending_fill.pop(c):
            cp.wait()

        pslot = c % 2
        xs0, xs1 = (2 * c) % _NSLOT, (2 * c + 1) % _NSLOT

        @plsc.parallel_loop(0, _NG * _DW, unroll=8)
        def _transpose_add(i):
            g = i // _DW
            d = i % _DW
            base = g * 16
            row = base + iota
            col = jnp.full((16,), d, jnp.int32)
            vals = plsc.load_gather(pe_v.at[pslot], [row, col])
            plsc.addupdate(x_v.at[xs0, d, pl.ds(base, 16)], vals)
            plsc.addupdate(x_v.at[xs1, d, pl.ds(base, 16)], vals)

        pending_drain.update(drain(c))

    for slot in sorted(pending_drain):
        pending_drain.pop(slot).wait()


def kernel(x, pe_table):
    mesh = plsc.VectorSubcoreMesh(core_axis_name="c", subcore_axis_name="s")
    k = functools.partial(
        pl.kernel,
        mesh=mesh,
        compiler_params=pltpu.CompilerParams(needs_layout_passes=False),
        out_type=jax.ShapeDtypeStruct((_B, _D, _S), jnp.float32),
        scratch_types=[
            pltpu.VMEM((2, _SC, _DW), jnp.float32),
            pltpu.VMEM((_NSLOT, _DW, _SC), jnp.float32),
            [pltpu.SemaphoreType.DMA, pltpu.SemaphoreType.DMA],
            [pltpu.SemaphoreType.DMA] * _NSLOT,
            [pltpu.SemaphoreType.DMA] * _NSLOT,
        ],
    )(_sc_body)
    return k(x, pe_table)


# speedup vs baseline: 1.7117x; 1.7117x over previous
import functools

import jax
import jax.numpy as jnp
from jax import lax
from jax.experimental import pallas as pl
from jax.experimental.pallas import tpu as pltpu
from jax.experimental.pallas import tpu_sc as plsc

_B = 2
_D = 1024
_S = 8192
_NWD = 8
_NWS = 4
_DW = _D // _NWD
_SW = _S // _NWS
_SC = 128
_NCHUNK = _SW // _SC
_NG = _SC // 16
_NSLOT = 5


def _sc_body(x_hbm, pe_hbm, o_hbm, pe_v, x_v, sem_pe, sem_x, sem_out):
    wid = lax.axis_index("s") * 2 + lax.axis_index("c")
    d0 = (wid % _NWD) * _DW
    s_base = (wid // _NWD) * _SW
    iota = lax.iota(jnp.int32, 16)

    def fill(c):
        s0 = s_base + c * _SC
        pslot = c % 2
        xs0, xs1 = (2 * c) % _NSLOT, (2 * c + 1) % _NSLOT
        return (
            pltpu.async_copy(pe_hbm.at[pl.ds(s0, _SC), pl.ds(d0, _DW)],
                             pe_v.at[pslot], sem_pe[pslot]),
            pltpu.async_copy(x_hbm.at[0, pl.ds(d0, _DW), pl.ds(s0, _SC)],
                             x_v.at[xs0], sem_x[xs0]),
            pltpu.async_copy(x_hbm.at[1, pl.ds(d0, _DW), pl.ds(s0, _SC)],
                             x_v.at[xs1], sem_x[xs1]),
        )

    def drain(c):
        s0 = s_base + c * _SC
        xs0, xs1 = (2 * c) % _NSLOT, (2 * c + 1) % _NSLOT
        return {
            xs0: pltpu.async_copy(
                x_v.at[xs0], o_hbm.at[0, pl.ds(d0, _DW), pl.ds(s0, _SC)],
                sem_out[xs0]),
            xs1: pltpu.async_copy(
                x_v.at[xs1], o_hbm.at[1, pl.ds(d0, _DW), pl.ds(s0, _SC)],
                sem_out[xs1]),
        }

    pending_fill = {0: fill(0)}
    pending_drain = {}

    for c in range(_NCHUNK):
        if c + 1 < _NCHUNK:
            for slot in ((2 * (c + 1)) % _NSLOT, (2 * (c + 1) + 1) % _NSLOT):
                if slot in pending_drain:
                    pending_drain.pop(slot).wait()
            pending_fill[c + 1] = fill(c + 1)
        for cp in pending_fill.pop(c):
            cp.wait()

        pslot = c % 2
        xs0, xs1 = (2 * c) % _NSLOT, (2 * c + 1) % _NSLOT

        @plsc.parallel_loop(0, (_SC // 16) * (_DW // 16), unroll=1)
        def _transpose_add(bi):
            sb = (bi // (_DW // 16)) * 16
            db = (bi % (_DW // 16)) * 16
            svec = sb + iota
            for r in range(16):
                rot = lax.bitwise_and(iota + r, 15)
                dvec = db + rot
                vals = plsc.load_gather(pe_v.at[pslot], [svec, dvec])
                plsc.addupdate_scatter(x_v.at[xs0], [dvec, svec], vals)
                plsc.addupdate_scatter(x_v.at[xs1], [dvec, svec], vals)

        pending_drain.update(drain(c))

    for slot in sorted(pending_drain):
        pending_drain.pop(slot).wait()


def kernel(x, pe_table):
    mesh = plsc.VectorSubcoreMesh(core_axis_name="c", subcore_axis_name="s")
    k = functools.partial(
        pl.kernel,
        mesh=mesh,
        compiler_params=pltpu.CompilerParams(needs_layout_passes=False),
        out_type=jax.ShapeDtypeStruct((_B, _D, _S), jnp.float32),
        scratch_types=[
            pltpu.VMEM((2, _SC, _DW), jnp.float32),
            pltpu.VMEM((_NSLOT, _DW, _SC), jnp.float32),
            [pltpu.SemaphoreType.DMA, pltpu.SemaphoreType.DMA],
            [pltpu.SemaphoreType.DMA] * _NSLOT,
            [pltpu.SemaphoreType.DMA] * _NSLOT,
        ],
    )(_sc_body)
    return k(x, pe_table)
